# baseline (speedup 1.0000x reference)
import functools

import jax
import jax.numpy as jnp
from jax import lax
from jax.experimental import pallas as pl
from jax.experimental.pallas import tpu as pltpu
from jax.experimental.pallas import tpu_sc as plsc

VOCAB = 1000
EMBED = 32
BATCH = 4096
HIST = 200

B_SC = 2560
N_SC = B_SC * HIST
N_TC = (BATCH - B_SC) * HIST

LANE = 128
ROWS_SC = N_SC // LANE
NWORKERS = 32
RPW = ROWS_SC // NWORKERS
IPW = RPW * LANE
CH = 5
NCHUNK = RPW // CH
NS = 4

_mesh = plsc.VectorSubcoreMesh(core_axis_name="c", subcore_axis_name="s")


@functools.partial(
    pl.kernel,
    mesh=_mesh,
    out_type=jax.ShapeDtypeStruct((N_SC, EMBED), jnp.float32),
    scratch_types=[
        pltpu.VMEM((RPW, LANE), jnp.int32),
        pltpu.VMEM((NS, CH * LANE, EMBED), jnp.float32),
        pltpu.VMEM_SHARED((VOCAB, EMBED), jnp.float32),
        pltpu.SemaphoreType.DMA((NS,)),
        pltpu.SemaphoreType.DMA((NS,)),
    ],
    compiler_params=pltpu.CompilerParams(use_tc_tiling_on_sc=False),
)
def _sc_gather(idx_hbm, table_hbm, out_hbm, idx_v, ring, table_sh,
               gat_sems, out_sems):
    sid = lax.axis_index("s")
    wid = sid * 2 + lax.axis_index("c")
    ibase = wid * IPW

    @pl.when(sid == 0)
    def _stage_table():
        pltpu.sync_copy(table_hbm, table_sh)

    pltpu.sync_copy(idx_hbm.at[pl.ds(wid * RPW, RPW)], idx_v)
    plsc.subcore_barrier()

    def fire_gathers(j):
        s = j % NS
        return [
            pltpu.async_copy(
                table_sh.at[idx_v.at[j * CH + k]],
                ring.at[s].at[pl.ds(k * LANE, LANE)],
                gat_sems.at[s],
            )
            for k in range(CH)
        ]

    out_handles = [None] * NCHUNK
    gat_handles = fire_gathers(0)
    for j in range(NCHUNK):
        if j + 1 < NCHUNK:
            if j + 1 >= NS:
                out_handles[j + 1 - NS].wait()
            next_handles = fire_gathers(j + 1)
        else:
            next_handles = None
        for h in gat_handles:
            h.wait()
        out_handles[j] = pltpu.async_copy(
            ring.at[j % NS],
            out_hbm.at[pl.ds(ibase + j * CH * LANE, CH * LANE)],
            out_sems.at[j % NS],
        )
        gat_handles = next_handles
    for j in range(NCHUNK - NS, NCHUNK):
        out_handles[j].wait()


BT = 1024
VPAD = 1024


def _tc_body(idx_ref, w_ref, out_ref):
    idxb = idx_ref[0, 0]
    onehot = (
        idxb[:, None] == lax.broadcasted_iota(jnp.int32, (BT, VPAD), 1)
    ).astype(jnp.float32)
    out_ref[...] = jnp.dot(onehot, w_ref[...],
                           preferred_element_type=jnp.float32)


_tc_gather = pl.pallas_call(
    _tc_body,
    grid=(N_TC // BT,),
    in_specs=[
        pl.BlockSpec((1, 1, BT), lambda i: (i, 0, 0)),
        pl.BlockSpec((VPAD, EMBED), lambda i: (0, 0)),
    ],
    out_specs=pl.BlockSpec((BT, EMBED), lambda i: (i, 0)),
    out_shape=jax.ShapeDtypeStruct((N_TC, EMBED), jnp.float32),
)


def kernel(word_idx_list, W):
    idx = word_idx_list.astype(jnp.int32).reshape(BATCH * HIST)
    idx_sc = idx[:N_SC].reshape(ROWS_SC, LANE)
    idx_tc = idx[N_SC:].reshape(N_TC // BT, 1, BT)
    w_pad = jnp.zeros((VPAD, EMBED), jnp.float32).at[:VOCAB].set(W)
    out_sc = _sc_gather(idx_sc, W)
    out_tc = _tc_gather(idx_tc, w_pad)
    out = jnp.concatenate([out_sc, out_tc], axis=0)
    return out.reshape(BATCH, HIST, EMBED)


# speedup vs baseline: 2.1405x; 2.1405x over previous
import functools

import jax
import jax.numpy as jnp
from jax import lax
from jax.experimental import pallas as pl
from jax.experimental.pallas import tpu as pltpu
from jax.experimental.pallas import tpu_sc as plsc

VOCAB = 1000
EMBED = 32
BATCH = 4096
HIST = 200

DLEN = 512
N = BATCH * HIST
ROWS = N // DLEN
NWORKERS = 32
RPW = ROWS // NWORKERS
IPW = RPW * DLEN
CH = 1
NCHUNK = RPW // CH
NS = 4

_mesh = plsc.VectorSubcoreMesh(core_axis_name="c", subcore_axis_name="s")


@functools.partial(
    pl.kernel,
    mesh=_mesh,
    out_type=jax.ShapeDtypeStruct((N, EMBED), jnp.float32),
    scratch_types=[
        pltpu.VMEM((RPW, DLEN), jnp.int32),
        pltpu.VMEM((NS, CH * DLEN, EMBED), jnp.float32),
        pltpu.VMEM_SHARED((VOCAB, EMBED), jnp.float32),
        pltpu.SemaphoreType.DMA((NS,)),
        pltpu.SemaphoreType.DMA((NS,)),
    ],
    compiler_params=pltpu.CompilerParams(use_tc_tiling_on_sc=False),
)
def _gather_kernel(idx_hbm, table_hbm, out_hbm, idx_v, ring, table_sh,
                   gat_sems, out_sems):
    sid = lax.axis_index("s")
    wid = sid * 2 + lax.axis_index("c")
    ibase = wid * IPW

    @pl.when(sid == 0)
    def _stage_table():
        pltpu.sync_copy(table_hbm, table_sh)

    pltpu.sync_copy(idx_hbm.at[pl.ds(wid * RPW, RPW)], idx_v)
    plsc.subcore_barrier()

    def fire_gathers(j):
        s = j % NS
        return [
            pltpu.async_copy(
                table_sh.at[idx_v.at[j * CH + k]],
                ring.at[s].at[pl.ds(k * DLEN, DLEN)],
                gat_sems.at[s],
            )
            for k in range(CH)
        ]

    out_handles = [None] * NCHUNK
    gat_handles = fire_gathers(0)
    for j in range(NCHUNK):
        if j + 1 < NCHUNK:
            if j + 1 >= NS:
                out_handles[j + 1 - NS].wait()
            next_handles = fire_gathers(j + 1)
        else:
            next_handles = None
        for h in gat_handles:
            h.wait()
        out_handles[j] = pltpu.async_copy(
            ring.at[j % NS],
            out_hbm.at[pl.ds(ibase + j * CH * DLEN, CH * DLEN)],
            out_sems.at[j % NS],
        )
        gat_handles = next_handles
    for j in range(NCHUNK - NS, NCHUNK):
        out_handles[j].wait()


def kernel(word_idx_list, W):
    idx = word_idx_list.astype(jnp.int32).reshape(ROWS, DLEN)
    out = _gather_kernel(idx, W)
    return out.reshape(BATCH, HIST, EMBED)
